# baseline (speedup 1.0000x reference)
import functools

import jax
import jax.numpy as jnp
from jax import lax
from jax.experimental import pallas as pl
from jax.experimental.pallas import tpu as pltpu
from jax.experimental.pallas import tpu_sc as plsc

_NS = 16
_S = 32
_KP = 3


def _topk_block(p1t_ref, p2_ref, idx_ref):
    R = p2_ref.shape[1]
    N = p1t_ref.shape[2]
    G = N // _S
    p2 = p2_ref[0]
    d = None
    for i in range(3):
        t = p2[:, i : i + 1] - p1t_ref[0, i : i + 1, :]
        d = t * t if d is None else d + t * t
    d3 = d.reshape(R, _S, G)

    big = jnp.float32(3.0e38)
    iota_s = jax.lax.broadcasted_iota(jnp.int32, (R, _S, G), 1)

    ms, ss = [], []
    for _ in range(_KP):
        mk = jnp.min(d3, axis=1)
        eqk = d3 <= mk[:, None, :]
        sk = jnp.min(jnp.where(eqk, iota_s, _S), axis=1)
        d3 = jnp.where(eqk, big, d3)
        ms.append(mk)
        ss.append(sk)

    iota_g = jax.lax.broadcasted_iota(jnp.int32, (R, G), 1)
    cv = jnp.concatenate(ms, axis=1)
    ci = jnp.concatenate([s * G + iota_g for s in ss], axis=1)
    cif = ci.astype(jnp.float32)

    rows = []
    for _ in range(_NS):
        mn = jnp.min(cv, axis=1, keepdims=True)
        sel = cv <= mn
        rows.append(jnp.sum(jnp.where(sel, cif, 0.0), axis=1)[None, :])
        cv = jnp.where(sel, big, cv)
    idx_ref[0] = jnp.concatenate(rows, axis=0).astype(jnp.int32)


def _tc_topk_indices(p1, p2):
    B, N, _ = p1.shape
    M = p2.shape[1]
    R = min(128, M)
    p1t = jnp.zeros((B, 8, N), jnp.float32).at[:, :3, :].set(
        jnp.transpose(p1, (0, 2, 1))
    )
    p2p = jnp.zeros((B, M, 8), jnp.float32).at[:, :, :3].set(p2)
    grid = (B, M // R)
    idx = pl.pallas_call(
        _topk_block,
        grid=grid,
        in_specs=[
            pl.BlockSpec((1, 8, N), lambda b, m: (b, 0, 0)),
            pl.BlockSpec((1, R, 8), lambda b, m: (b, m, 0)),
        ],
        out_specs=pl.BlockSpec((1, _NS, R), lambda b, m: (b, 0, m)),
        out_shape=jax.ShapeDtypeStruct((B, _NS, M), jnp.int32),
    )(p1t, p2p)
    return idx


def _make_sc_gather_mean(BM, C, NW):
    QPW = BM // NW
    CHQ = 8
    NCH = QPW // CHQ

    @functools.partial(
        pl.kernel,
        mesh=plsc.VectorSubcoreMesh(core_axis_name="c", subcore_axis_name="s"),
        compiler_params=pltpu.CompilerParams(use_tc_tiling_on_sc=False),
        out_type=jax.ShapeDtypeStruct((BM, C), jnp.float32),
        scratch_types=[
            pltpu.VMEM((CHQ * _NS,), jnp.int32),
            pltpu.VMEM((CHQ * _NS, C), jnp.float32),
            pltpu.VMEM((QPW, C), jnp.float32),
            pltpu.SemaphoreType.DMA,
        ],
    )
    def sc_fn(x1t_hbm, idx_hbm, out_hbm, idxb, rows, outbuf, sem):
        wid = lax.axis_index("s") * 2 + lax.axis_index("c")
        base_q = wid * QPW

        def chunk_body(ch, carry):
            q0 = (base_q + ch * CHQ) * _NS
            pltpu.sync_copy(idx_hbm.at[pl.ds(q0, CHQ * _NS)], idxb)
            pltpu.async_copy(x1t_hbm.at[idxb], rows, sem).wait()

            def q_body(qq, carry2):
                def j_body(j, acc):
                    r = qq * _NS + j
                    return tuple(
                        acc[v] + rows[r, pl.ds(v * 16, 16)]
                        for v in range(C // 16)
                    )

                acc = lax.fori_loop(
                    0, _NS, j_body,
                    tuple(jnp.zeros((16,), jnp.float32)
                          for _ in range(C // 16)),
                )
                for v in range(C // 16):
                    outbuf[ch * CHQ + qq, pl.ds(v * 16, 16)] = (
                        acc[v] * (1.0 / _NS)
                    )
                return carry2

            lax.fori_loop(0, CHQ, q_body, 0)
            return carry

        lax.fori_loop(0, NCH, chunk_body, 0)
        pltpu.sync_copy(outbuf, out_hbm.at[pl.ds(base_q, QPW)])

    return sc_fn


@jax.jit
def kernel(p1, x1, p2):
    B, N, _ = p1.shape
    C = x1.shape[1]
    M = p2.shape[1]
    idx = _tc_topk_indices(p1, p2)
    idx_t = jnp.transpose(idx, (0, 2, 1))
    idx_g = idx_t + (jnp.arange(B, dtype=jnp.int32) * N)[:, None, None]
    idx_flat = idx_g.reshape(B * M * _NS)

    x1t = jnp.transpose(x1, (0, 2, 1)).reshape(B * N, C)
    out = _make_sc_gather_mean(B * M, C, 32)(x1t, idx_flat)
    return jnp.transpose(out.reshape(B, M, C), (0, 2, 1))


# speedup vs baseline: 1.0296x; 1.0296x over previous
import functools

import jax
import jax.numpy as jnp
from jax import lax
from jax.experimental import pallas as pl
from jax.experimental.pallas import tpu as pltpu
from jax.experimental.pallas import tpu_sc as plsc

_NS = 16
_S = 32
_KP = 3


def _topk_block(p1t_ref, p2_ref, idx_ref):
    R = p2_ref.shape[1]
    N = p1t_ref.shape[2]
    G = N // _S
    p2 = p2_ref[0]
    d = jnp.broadcast_to(p1t_ref[0, 3:4, :], (R, N))
    for i in range(3):
        d = d + p2[:, i : i + 1] * p1t_ref[0, i : i + 1, :]
    d3 = d.reshape(R, _S, G)

    big = jnp.float32(3.0e38)
    iota_s = jax.lax.broadcasted_iota(jnp.int32, (R, _S, G), 1)

    ms, ss = [], []
    for _ in range(_KP):
        mk = jnp.min(d3, axis=1)
        eqk = d3 <= mk[:, None, :]
        sk = jnp.min(jnp.where(eqk, iota_s, _S), axis=1)
        d3 = jnp.where(eqk, big, d3)
        ms.append(mk)
        ss.append(sk)

    iota_g = jax.lax.broadcasted_iota(jnp.int32, (R, G), 1)
    cv = jnp.concatenate(ms, axis=1)
    ci = jnp.concatenate([s * G + iota_g for s in ss], axis=1)
    cif = ci.astype(jnp.float32)

    rows = []
    for _ in range(_NS):
        mn = jnp.min(cv, axis=1, keepdims=True)
        sel = cv <= mn
        rows.append(jnp.sum(jnp.where(sel, cif, 0.0), axis=1)[None, :])
        cv = jnp.where(sel, big, cv)
    idx_ref[0] = jnp.concatenate(rows, axis=0).astype(jnp.int32)


def _tc_topk_indices(p1, p2):
    B, N, _ = p1.shape
    M = p2.shape[1]
    R = min(128, M)
    p1c = jnp.transpose(p1, (0, 2, 1))
    p1t = (
        jnp.zeros((B, 8, N), jnp.float32)
        .at[:, :3, :].set(-2.0 * p1c)
        .at[:, 3, :].set(jnp.sum(p1c * p1c, axis=1))
    )
    p2p = jnp.zeros((B, M, 8), jnp.float32).at[:, :, :3].set(p2)
    grid = (B, M // R)
    idx = pl.pallas_call(
        _topk_block,
        grid=grid,
        in_specs=[
            pl.BlockSpec((1, 8, N), lambda b, m: (b, 0, 0)),
            pl.BlockSpec((1, R, 8), lambda b, m: (b, m, 0)),
        ],
        out_specs=pl.BlockSpec((1, _NS, R), lambda b, m: (b, 0, m)),
        out_shape=jax.ShapeDtypeStruct((B, _NS, M), jnp.int32),
    )(p1t, p2p)
    return idx


def _make_sc_gather_mean(BM, C, NW):
    QPW = BM // NW
    CHQ = 8
    NCH = QPW // CHQ

    @functools.partial(
        pl.kernel,
        mesh=plsc.VectorSubcoreMesh(core_axis_name="c", subcore_axis_name="s"),
        compiler_params=pltpu.CompilerParams(use_tc_tiling_on_sc=False),
        out_type=jax.ShapeDtypeStruct((BM, C), jnp.float32),
        scratch_types=[
            pltpu.VMEM((CHQ * _NS,), jnp.int32),
            pltpu.VMEM((CHQ * _NS, C), jnp.float32),
            pltpu.VMEM((QPW, C), jnp.float32),
            pltpu.SemaphoreType.DMA,
        ],
    )
    def sc_fn(x1t_hbm, idx_hbm, out_hbm, idxb, rows, outbuf, sem):
        wid = lax.axis_index("s") * 2 + lax.axis_index("c")
        base_q = wid * QPW

        def chunk_body(ch, carry):
            q0 = (base_q + ch * CHQ) * _NS
            pltpu.sync_copy(idx_hbm.at[pl.ds(q0, CHQ * _NS)], idxb)
            pltpu.async_copy(x1t_hbm.at[idxb], rows, sem).wait()

            def q_body(qq, carry2):
                def j_body(j, acc):
                    r = qq * _NS + j
                    return tuple(
                        acc[v] + rows[r, pl.ds(v * 16, 16)]
                        for v in range(C // 16)
                    )

                acc = lax.fori_loop(
                    0, _NS, j_body,
                    tuple(jnp.zeros((16,), jnp.float32)
                          for _ in range(C // 16)),
                )
                for v in range(C // 16):
                    outbuf[ch * CHQ + qq, pl.ds(v * 16, 16)] = (
                        acc[v] * (1.0 / _NS)
                    )
                return carry2

            lax.fori_loop(0, CHQ, q_body, 0)
            return carry

        lax.fori_loop(0, NCH, chunk_body, 0)
        pltpu.sync_copy(outbuf, out_hbm.at[pl.ds(base_q, QPW)])

    return sc_fn


@jax.jit
def kernel(p1, x1, p2):
    B, N, _ = p1.shape
    C = x1.shape[1]
    M = p2.shape[1]
    idx = _tc_topk_indices(p1, p2)
    idx_t = jnp.transpose(idx, (0, 2, 1))
    idx_g = idx_t + (jnp.arange(B, dtype=jnp.int32) * N)[:, None, None]
    idx_flat = idx_g.reshape(B * M * _NS)

    x1t = jnp.transpose(x1, (0, 2, 1)).reshape(B * N, C)
    out = _make_sc_gather_mean(B * M, C, 32)(x1t, idx_flat)
    return jnp.transpose(out.reshape(B, M, C), (0, 2, 1))
